# baseline (speedup 1.0000x reference)
import functools

import jax
import jax.numpy as jnp
from jax import lax
from jax.experimental import pallas as pl
from jax.experimental.pallas import tpu as pltpu
from jax.experimental.pallas import tpu_sc as plsc

N = 10000
E = 320000
D_IN, H1, H2 = 128, 64, 32

NC, NS = 2, 16
NW = NC * NS

NP = 10240
RPT = NP // NS
IW = 256
CHUNK = 256
KSUB = CHUNK // IW
EP = 327680
NCHUNK = EP // CHUNK
CH0 = 64
CH1 = 16
RPWMAX = CH0 * KSUB
EPL = (NS * CH0 + NS * CH1 + RPWMAX) * CHUNK

_MESH = plsc.VectorSubcoreMesh(core_axis_name="c", subcore_axis_name="s",
                               num_cores=NC, num_subcores=NS)
_SC_PARAMS = pltpu.CompilerParams(use_tc_tiling_on_sc=False)


DRPW = EP // IW // NW

@functools.partial(
    pl.kernel,
    out_type=jax.ShapeDtypeStruct((NC, NP), jnp.float32),
    mesh=_MESH,
    scratch_types=[
        pltpu.VMEM((DRPW, IW), jnp.int32),
        pltpu.VMEM((IW,), jnp.float32),
        pltpu.VMEM((RPT,), jnp.float32),
        pltpu.VMEM_SHARED((NP,), jnp.float32),
    ],
    compiler_params=_SC_PARAMS,
)
def _deg_kernel(dst_hbm, out_hbm, didx, ones, zbuf, acc):
    cid = lax.axis_index("c")
    sid = lax.axis_index("s")
    wid = cid * NS + sid
    zero16 = jnp.zeros((16,), jnp.float32)
    one16 = jnp.ones((16,), jnp.float32)

    def _zero(i, _):
        zbuf[pl.ds(i * 16, 16)] = zero16
        return 0

    lax.fori_loop(0, RPT // 16, _zero, 0)
    for i in range(IW // 16):
        ones[pl.ds(i * 16, 16)] = one16
    pltpu.sync_copy(zbuf, acc.at[pl.ds(sid * RPT, RPT)])
    plsc.subcore_barrier()

    pltpu.sync_copy(dst_hbm.at[pl.ds(wid * DRPW, DRPW)], didx)

    def _body(k, _):
        pltpu.sync_copy(ones, acc.at[didx.at[k]], add=True)
        return 0

    lax.fori_loop(0, DRPW, _body, 0)
    plsc.subcore_barrier()
    pltpu.sync_copy(acc.at[pl.ds(sid * RPT, RPT)],
                    out_hbm.at[cid, pl.ds(sid * RPT, RPT)])


def _make_agg(H):
    @functools.partial(
        pl.kernel,
        out_type=jax.ShapeDtypeStruct((NC, NP, H), jnp.float32),
        mesh=_MESH,
        scratch_types=[
            pltpu.VMEM((RPWMAX, IW), jnp.int32),
            pltpu.VMEM((RPWMAX, IW), jnp.int32),
            pltpu.VMEM((2, CHUNK, H), jnp.float32),
            pltpu.VMEM((32, H), jnp.float32),
            pltpu.VMEM_SHARED((NP, H), jnp.float32),
            pltpu.SemaphoreType.DMA,
            pltpu.SemaphoreType.DMA,
            pltpu.SemaphoreType.DMA,
            pltpu.SemaphoreType.DMA,
        ],
        compiler_params=_SC_PARAMS,
    )
    def _agg(hs_hbm, src_hbm, dst_hbm, out_hbm, sidx, didx, rows, zbuf, acc,
             g0, g1, s0, s1):
        cid = lax.axis_index("c")
        sid = lax.axis_index("s")
        zero16 = jnp.zeros((16,), jnp.float32)
        hh = H // 16

        def _zero(i, _):
            zbuf[i // hh, pl.ds((i % hh) * 16, 16)] = zero16
            return 0

        lax.fori_loop(0, 32 * hh, _zero, 0)

        def _init(t, _):
            pltpu.sync_copy(zbuf, acc.at[pl.ds(sid * RPT + t * 32, 32)])
            return 0

        lax.fori_loop(0, RPT // 32, _init, 0)
        plsc.subcore_barrier()

        cpw = jnp.where(cid == 0, CH0, CH1)
        ib = jnp.where(cid == 0, sid * (CH0 * KSUB),
                       NS * (CH0 * KSUB) + sid * (CH1 * KSUB))
        pltpu.sync_copy(src_hbm.at[pl.ds(ib, RPWMAX)], sidx)
        pltpu.sync_copy(dst_hbm.at[pl.ds(ib, RPWMAX)], didx)

        rb = (rows.at[0], rows.at[1])
        gs = (g0, g1)
        ss = (s0, s1)

        def fire_g(k, b):
            pltpu.async_copy(hs_hbm.at[sidx.at[k]], rb[b], gs[b])

        def wait_g(b):
            pltpu.make_async_copy(hs_hbm.at[pl.ds(0, CHUNK)], rb[b],
                                  gs[b]).wait()

        def fire_s(k, b):
            pltpu.async_copy(rb[b], acc.at[didx.at[k]], ss[b], add=True)

        def wait_s(k, b):
            pltpu.make_async_copy(rb[b], acc.at[didx.at[k]], ss[b]).wait()

        fire_g(0, 0)
        fire_g(1, 1)
        wait_g(0)
        fire_s(0, 0)

        def _body(i, _):
            k1 = 2 * i + 1
            wait_g(1)
            fire_s(k1, 1)
            wait_s(k1 - 1, 0)
            fire_g(k1 + 1, 0)
            k2 = 2 * i + 2
            wait_g(0)
            fire_s(k2, 0)
            wait_s(k1, 1)
            fire_g(k2 + 1, 1)
            return 0

        lax.fori_loop(0, (cpw - 2) // 2, _body, 0)
        klast = cpw - 1
        wait_g(1)
        fire_s(klast, 1)
        wait_s(klast - 1, 0)
        wait_s(klast, 1)
        plsc.subcore_barrier()
        for t in range(RPT // 128):
            r0 = sid * RPT + t * 128
            pltpu.sync_copy(acc.at[pl.ds(r0, 128)],
                            out_hbm.at[cid, pl.ds(r0, 128)])

    return _agg


_agg_h1 = _make_agg(H1)
_agg_h2 = _make_agg(H2)


RB = 1000

def _dinv_from(deg_blk):
    d = deg_blk[:, 0:1] + deg_blk[:, 1:2] + 1.0
    return lax.rsqrt(d)


def _tc_a_body(deg_ref, x_ref, w1_ref, hs_ref):
    dinv = _dinv_from(deg_ref[...])
    h = jnp.dot(x_ref[...], w1_ref[...], preferred_element_type=jnp.float32)
    hs_ref[...] = h * dinv


def _tc_b_body(deg_ref, agg_ref, hs1_ref, w2_ref, b1_ref, hs2_ref):
    dinv = _dinv_from(deg_ref[...])
    agg = agg_ref[0] + agg_ref[1]
    z = jnp.maximum(dinv * (agg + hs1_ref[...]) + b1_ref[...], 0.0)
    h2 = jnp.dot(z, w2_ref[...], preferred_element_type=jnp.float32)
    hs2_ref[...] = h2 * dinv


def _tc_c_body(deg_ref, agg_ref, hs2_ref, wfc_ref, b2_ref, bfc_ref, out_ref):
    dinv = _dinv_from(deg_ref[...])
    agg = agg_ref[0] + agg_ref[1]
    z = jnp.maximum(dinv * (agg + hs2_ref[...]) + b2_ref[...], 0.0)
    o = jnp.dot(z, wfc_ref[...], preferred_element_type=jnp.float32)
    out_ref[...] = 1.0 / (1.0 + jnp.exp(-(o + bfc_ref[...])))


_G = N // RB

_tc_a = pl.pallas_call(
    _tc_a_body,
    grid=(_G,),
    in_specs=[
        pl.BlockSpec((RB, 2), lambda i: (i, 0)),
        pl.BlockSpec((RB, D_IN), lambda i: (i, 0)),
        pl.BlockSpec((D_IN, H1), lambda i: (0, 0)),
    ],
    out_specs=pl.BlockSpec((RB, H1), lambda i: (i, 0)),
    out_shape=jax.ShapeDtypeStruct((N, H1), jnp.float32),
)

_tc_b = pl.pallas_call(
    _tc_b_body,
    grid=(_G,),
    in_specs=[
        pl.BlockSpec((RB, 2), lambda i: (i, 0)),
        pl.BlockSpec((NC, RB, H1), lambda i: (0, i, 0)),
        pl.BlockSpec((RB, H1), lambda i: (i, 0)),
        pl.BlockSpec((H1, H2), lambda i: (0, 0)),
        pl.BlockSpec((1, H1), lambda i: (0, 0)),
    ],
    out_specs=pl.BlockSpec((RB, H2), lambda i: (i, 0)),
    out_shape=jax.ShapeDtypeStruct((N, H2), jnp.float32),
)

_tc_c = pl.pallas_call(
    _tc_c_body,
    grid=(_G,),
    in_specs=[
        pl.BlockSpec((RB, 2), lambda i: (i, 0)),
        pl.BlockSpec((NC, RB, H2), lambda i: (0, i, 0)),
        pl.BlockSpec((RB, H2), lambda i: (i, 0)),
        pl.BlockSpec((H2, 1), lambda i: (0, 0)),
        pl.BlockSpec((1, H2), lambda i: (0, 0)),
        pl.BlockSpec((1, 1), lambda i: (0, 0)),
    ],
    out_specs=pl.BlockSpec((RB, 1), lambda i: (i, 0)),
    out_shape=jax.ShapeDtypeStruct((N, 1), jnp.float32),
)


def kernel(x, edge_index, W1, b1, W2, b2, Wfc, bfc):
    src = edge_index[0]
    dst = edge_index[1]
    pad_src = jnp.zeros((EPL - E,), jnp.int32)
    pad_dst = jnp.full((EPL - E,), N, jnp.int32)
    src2d = jnp.concatenate([src, pad_src]).reshape(EPL // IW, IW)
    dst2d = jnp.concatenate([dst, pad_dst]).reshape(EPL // IW, IW)

    degp = _deg_kernel(dst2d)
    degT = degp.T

    hs1 = _tc_a(degT, x, W1)
    agg1 = _agg_h1(hs1, src2d, dst2d)
    hs2 = _tc_b(degT, agg1, hs1, W2, b1.reshape(1, H1))
    agg2 = _agg_h2(hs2, src2d, dst2d)
    out = _tc_c(degT, agg2, hs2, Wfc, b2.reshape(1, H2), bfc.reshape(1, 1))
    return out


# speedup vs baseline: 2.0947x; 2.0947x over previous
import functools

import jax
import jax.numpy as jnp
from jax import lax
from jax.experimental import pallas as pl
from jax.experimental.pallas import tpu as pltpu
from jax.experimental.pallas import tpu_sc as plsc

N = 10000
E = 320000
D_IN, H1, H2 = 128, 64, 32

NC, NS = 2, 16
NW = NC * NS

NP = 10240
RPT = NP // NS
IW = 256
CHUNK = 256
KSUB = CHUNK // IW
EP = 327680
NCHUNK = EP // CHUNK
CH0 = 40
CH1 = 40
RPWMAX = CH0 * KSUB
EPL = (NS * CH0 + NS * CH1 + RPWMAX) * CHUNK

_MESH = plsc.VectorSubcoreMesh(core_axis_name="c", subcore_axis_name="s",
                               num_cores=NC, num_subcores=NS)
_SC_PARAMS = pltpu.CompilerParams(use_tc_tiling_on_sc=False)


DRPW = EP // IW // NW

@functools.partial(
    pl.kernel,
    out_type=jax.ShapeDtypeStruct((NC, NP), jnp.float32),
    mesh=_MESH,
    scratch_types=[
        pltpu.VMEM((DRPW, IW), jnp.int32),
        pltpu.VMEM((IW,), jnp.float32),
        pltpu.VMEM((RPT,), jnp.float32),
        pltpu.VMEM_SHARED((NP,), jnp.float32),
    ],
    compiler_params=_SC_PARAMS,
)
def _deg_kernel(dst_hbm, out_hbm, didx, ones, zbuf, acc):
    cid = lax.axis_index("c")
    sid = lax.axis_index("s")
    wid = cid * NS + sid
    zero16 = jnp.zeros((16,), jnp.float32)
    one16 = jnp.ones((16,), jnp.float32)

    def _zero(i, _):
        zbuf[pl.ds(i * 16, 16)] = zero16
        return 0

    lax.fori_loop(0, RPT // 16, _zero, 0)
    for i in range(IW // 16):
        ones[pl.ds(i * 16, 16)] = one16
    pltpu.sync_copy(zbuf, acc.at[pl.ds(sid * RPT, RPT)])
    plsc.subcore_barrier()

    pltpu.sync_copy(dst_hbm.at[pl.ds(wid * DRPW, DRPW)], didx)

    def _body(k, _):
        pltpu.sync_copy(ones, acc.at[didx.at[k]], add=True)
        return 0

    lax.fori_loop(0, DRPW, _body, 0)
    plsc.subcore_barrier()
    pltpu.sync_copy(acc.at[pl.ds(sid * RPT, RPT)],
                    out_hbm.at[cid, pl.ds(sid * RPT, RPT)])


def _make_agg(H):
    @functools.partial(
        pl.kernel,
        out_type=jax.ShapeDtypeStruct((NC, NP, H), jnp.float32),
        mesh=_MESH,
        scratch_types=[
            pltpu.VMEM((RPWMAX, IW), jnp.int32),
            pltpu.VMEM((RPWMAX, IW), jnp.int32),
            pltpu.VMEM((2, CHUNK, H), jnp.float32),
            pltpu.VMEM((32, H), jnp.float32),
            pltpu.VMEM_SHARED((NP, H), jnp.float32),
            pltpu.SemaphoreType.DMA,
            pltpu.SemaphoreType.DMA,
            pltpu.SemaphoreType.DMA,
            pltpu.SemaphoreType.DMA,
        ],
        compiler_params=_SC_PARAMS,
    )
    def _agg(hs_hbm, src_hbm, dst_hbm, out_hbm, sidx, didx, rows, zbuf, acc,
             g0, g1, s0, s1):
        cid = lax.axis_index("c")
        sid = lax.axis_index("s")
        zero16 = jnp.zeros((16,), jnp.float32)
        hh = H // 16

        def _zero(i, _):
            zbuf[i // hh, pl.ds((i % hh) * 16, 16)] = zero16
            return 0

        lax.fori_loop(0, 32 * hh, _zero, 0)

        def _init(t, _):
            pltpu.sync_copy(zbuf, acc.at[pl.ds(sid * RPT + t * 32, 32)])
            return 0

        lax.fori_loop(0, RPT // 32, _init, 0)
        plsc.subcore_barrier()

        cpw = jnp.where(cid == 0, CH0, CH1)
        ib = jnp.where(cid == 0, sid * (CH0 * KSUB),
                       NS * (CH0 * KSUB) + sid * (CH1 * KSUB))
        pltpu.sync_copy(src_hbm.at[pl.ds(ib, RPWMAX)], sidx)
        pltpu.sync_copy(dst_hbm.at[pl.ds(ib, RPWMAX)], didx)

        rb = (rows.at[0], rows.at[1])
        gs = (g0, g1)
        ss = (s0, s1)

        def fire_g(k, b):
            pltpu.async_copy(hs_hbm.at[sidx.at[k]], rb[b], gs[b])

        def wait_g(b):
            pltpu.make_async_copy(hs_hbm.at[pl.ds(0, CHUNK)], rb[b],
                                  gs[b]).wait()

        def fire_s(k, b):
            pltpu.async_copy(rb[b], acc.at[didx.at[k]], ss[b], add=True)

        def wait_s(k, b):
            pltpu.make_async_copy(rb[b], acc.at[didx.at[k]], ss[b]).wait()

        fire_g(0, 0)
        fire_g(1, 1)
        wait_g(0)
        fire_s(0, 0)

        def _body(i, _):
            k1 = 2 * i + 1
            wait_g(1)
            fire_s(k1, 1)
            wait_s(k1 - 1, 0)
            fire_g(k1 + 1, 0)
            k2 = 2 * i + 2
            wait_g(0)
            fire_s(k2, 0)
            wait_s(k1, 1)
            fire_g(k2 + 1, 1)
            return 0

        lax.fori_loop(0, (cpw - 2) // 2, _body, 0)
        klast = cpw - 1
        wait_g(1)
        fire_s(klast, 1)
        wait_s(klast - 1, 0)
        wait_s(klast, 1)
        plsc.subcore_barrier()
        for t in range(RPT // 128):
            r0 = sid * RPT + t * 128
            pltpu.sync_copy(acc.at[pl.ds(r0, 128)],
                            out_hbm.at[cid, pl.ds(r0, 128)])

    return _agg


_agg_h1 = _make_agg(H1)
_agg_h2 = _make_agg(H2)


RB = 1000

def _dinv_from(deg_blk):
    d = deg_blk[:, 0:1] + deg_blk[:, 1:2] + 1.0
    return lax.rsqrt(d)


def _tc_a_body(deg_ref, x_ref, w1_ref, hs_ref):
    dinv = _dinv_from(deg_ref[...])
    h = jnp.dot(x_ref[...], w1_ref[...], preferred_element_type=jnp.float32)
    hs_ref[...] = h * dinv


def _tc_b_body(deg_ref, agg_ref, hs1_ref, w2_ref, b1_ref, hs2_ref):
    dinv = _dinv_from(deg_ref[...])
    agg = agg_ref[0] + agg_ref[1]
    z = jnp.maximum(dinv * (agg + hs1_ref[...]) + b1_ref[...], 0.0)
    h2 = jnp.dot(z, w2_ref[...], preferred_element_type=jnp.float32)
    hs2_ref[...] = h2 * dinv


def _tc_c_body(deg_ref, agg_ref, hs2_ref, wfc_ref, b2_ref, bfc_ref, out_ref):
    dinv = _dinv_from(deg_ref[...])
    agg = agg_ref[0] + agg_ref[1]
    z = jnp.maximum(dinv * (agg + hs2_ref[...]) + b2_ref[...], 0.0)
    o = jnp.dot(z, wfc_ref[...], preferred_element_type=jnp.float32)
    out_ref[...] = 1.0 / (1.0 + jnp.exp(-(o + bfc_ref[...])))


_G = N // RB

_tc_a = pl.pallas_call(
    _tc_a_body,
    grid=(_G,),
    in_specs=[
        pl.BlockSpec((RB, 2), lambda i: (i, 0)),
        pl.BlockSpec((RB, D_IN), lambda i: (i, 0)),
        pl.BlockSpec((D_IN, H1), lambda i: (0, 0)),
    ],
    out_specs=pl.BlockSpec((RB, H1), lambda i: (i, 0)),
    out_shape=jax.ShapeDtypeStruct((N, H1), jnp.float32),
)

_tc_b = pl.pallas_call(
    _tc_b_body,
    grid=(_G,),
    in_specs=[
        pl.BlockSpec((RB, 2), lambda i: (i, 0)),
        pl.BlockSpec((NC, RB, H1), lambda i: (0, i, 0)),
        pl.BlockSpec((RB, H1), lambda i: (i, 0)),
        pl.BlockSpec((H1, H2), lambda i: (0, 0)),
        pl.BlockSpec((1, H1), lambda i: (0, 0)),
    ],
    out_specs=pl.BlockSpec((RB, H2), lambda i: (i, 0)),
    out_shape=jax.ShapeDtypeStruct((N, H2), jnp.float32),
)

_tc_c = pl.pallas_call(
    _tc_c_body,
    grid=(_G,),
    in_specs=[
        pl.BlockSpec((RB, 2), lambda i: (i, 0)),
        pl.BlockSpec((NC, RB, H2), lambda i: (0, i, 0)),
        pl.BlockSpec((RB, H2), lambda i: (i, 0)),
        pl.BlockSpec((H2, 1), lambda i: (0, 0)),
        pl.BlockSpec((1, H2), lambda i: (0, 0)),
        pl.BlockSpec((1, 1), lambda i: (0, 0)),
    ],
    out_specs=pl.BlockSpec((RB, 1), lambda i: (i, 0)),
    out_shape=jax.ShapeDtypeStruct((N, 1), jnp.float32),
)


def kernel(x, edge_index, W1, b1, W2, b2, Wfc, bfc):
    src = edge_index[0]
    dst = edge_index[1]
    pad_iota = jnp.arange(EPL - E, dtype=jnp.int32)
    pad_src = pad_iota % N
    pad_dst = N + pad_iota % (NP - N)
    src2d = jnp.concatenate([src, pad_src]).reshape(EPL // IW, IW)
    dst2d = jnp.concatenate([dst, pad_dst]).reshape(EPL // IW, IW)

    degp = _deg_kernel(dst2d)
    degT = degp.T

    hs1 = _tc_a(degT, x, W1)
    agg1 = _agg_h1(hs1, src2d, dst2d)
    hs2 = _tc_b(degT, agg1, hs1, W2, b1.reshape(1, H1))
    agg2 = _agg_h2(hs2, src2d, dst2d)
    out = _tc_c(degT, agg2, hs2, Wfc, b2.reshape(1, H2), bfc.reshape(1, 1))
    return out


# speedup vs baseline: 2.2465x; 1.0725x over previous
import functools

import jax
import jax.numpy as jnp
from jax import lax
from jax.experimental import pallas as pl
from jax.experimental.pallas import tpu as pltpu
from jax.experimental.pallas import tpu_sc as plsc

N = 10000
E = 320000
D_IN, H1, H2 = 128, 64, 32

NC, NS = 2, 16
NW = NC * NS

NP = 10240
RPT = NP // NS
IW = 512
CHUNK = 512
KSUB = CHUNK // IW
EP = 327680
NCHUNK = EP // CHUNK
CH0 = 20
CH1 = 20
RPWMAX = CH0 * KSUB
EPL = (NS * CH0 + NS * CH1 + RPWMAX) * CHUNK

_MESH = plsc.VectorSubcoreMesh(core_axis_name="c", subcore_axis_name="s",
                               num_cores=NC, num_subcores=NS)
_SC_PARAMS = pltpu.CompilerParams(use_tc_tiling_on_sc=False)


DRPW = EP // IW // NW

@functools.partial(
    pl.kernel,
    out_type=jax.ShapeDtypeStruct((NC, NP), jnp.float32),
    mesh=_MESH,
    scratch_types=[
        pltpu.VMEM((DRPW, IW), jnp.int32),
        pltpu.VMEM((IW,), jnp.float32),
        pltpu.VMEM((RPT,), jnp.float32),
        pltpu.VMEM_SHARED((NP,), jnp.float32),
    ],
    compiler_params=_SC_PARAMS,
)
def _deg_kernel(dst_hbm, out_hbm, didx, ones, zbuf, acc):
    cid = lax.axis_index("c")
    sid = lax.axis_index("s")
    wid = cid * NS + sid
    zero16 = jnp.zeros((16,), jnp.float32)
    one16 = jnp.ones((16,), jnp.float32)

    def _zero(i, _):
        zbuf[pl.ds(i * 16, 16)] = zero16
        return 0

    lax.fori_loop(0, RPT // 16, _zero, 0)
    for i in range(IW // 16):
        ones[pl.ds(i * 16, 16)] = one16
    pltpu.sync_copy(zbuf, acc.at[pl.ds(sid * RPT, RPT)])
    plsc.subcore_barrier()

    pltpu.sync_copy(dst_hbm.at[pl.ds(wid * DRPW, DRPW)], didx)

    def _body(k, _):
        pltpu.sync_copy(ones, acc.at[didx.at[k]], add=True)
        return 0

    lax.fori_loop(0, DRPW, _body, 0)
    plsc.subcore_barrier()
    pltpu.sync_copy(acc.at[pl.ds(sid * RPT, RPT)],
                    out_hbm.at[cid, pl.ds(sid * RPT, RPT)])


def _make_agg(H):
    @functools.partial(
        pl.kernel,
        out_type=jax.ShapeDtypeStruct((NC, NP, H), jnp.float32),
        mesh=_MESH,
        scratch_types=[
            pltpu.VMEM((RPWMAX, IW), jnp.int32),
            pltpu.VMEM((RPWMAX, IW), jnp.int32),
            pltpu.VMEM((2, CHUNK, H), jnp.float32),
            pltpu.VMEM((32, H), jnp.float32),
            pltpu.VMEM_SHARED((NP, H), jnp.float32),
            pltpu.SemaphoreType.DMA,
            pltpu.SemaphoreType.DMA,
            pltpu.SemaphoreType.DMA,
            pltpu.SemaphoreType.DMA,
        ],
        compiler_params=_SC_PARAMS,
    )
    def _agg(hs_hbm, src_hbm, dst_hbm, out_hbm, sidx, didx, rows, zbuf, acc,
             g0, g1, s0, s1):
        cid = lax.axis_index("c")
        sid = lax.axis_index("s")
        zero16 = jnp.zeros((16,), jnp.float32)
        hh = H // 16

        def _zero(i, _):
            zbuf[i // hh, pl.ds((i % hh) * 16, 16)] = zero16
            return 0

        lax.fori_loop(0, 32 * hh, _zero, 0)

        def _init(t, _):
            pltpu.sync_copy(zbuf, acc.at[pl.ds(sid * RPT + t * 32, 32)])
            return 0

        lax.fori_loop(0, RPT // 32, _init, 0)
        plsc.subcore_barrier()

        cpw = jnp.where(cid == 0, CH0, CH1)
        ib = jnp.where(cid == 0, sid * (CH0 * KSUB),
                       NS * (CH0 * KSUB) + sid * (CH1 * KSUB))
        pltpu.sync_copy(src_hbm.at[pl.ds(ib, RPWMAX)], sidx)
        pltpu.sync_copy(dst_hbm.at[pl.ds(ib, RPWMAX)], didx)

        rb = (rows.at[0], rows.at[1])
        gs = (g0, g1)
        ss = (s0, s1)

        def fire_g(k, b):
            pltpu.async_copy(hs_hbm.at[sidx.at[k]], rb[b], gs[b])

        def wait_g(b):
            pltpu.make_async_copy(hs_hbm.at[pl.ds(0, CHUNK)], rb[b],
                                  gs[b]).wait()

        def fire_s(k, b):
            pltpu.async_copy(rb[b], acc.at[didx.at[k]], ss[b], add=True)

        def wait_s(k, b):
            pltpu.make_async_copy(rb[b], acc.at[didx.at[k]], ss[b]).wait()

        fire_g(0, 0)
        fire_g(1, 1)
        wait_g(0)
        fire_s(0, 0)

        def _body(i, _):
            k1 = 2 * i + 1
            wait_g(1)
            fire_s(k1, 1)
            wait_s(k1 - 1, 0)
            fire_g(k1 + 1, 0)
            k2 = 2 * i + 2
            wait_g(0)
            fire_s(k2, 0)
            wait_s(k1, 1)
            fire_g(k2 + 1, 1)
            return 0

        lax.fori_loop(0, (cpw - 2) // 2, _body, 0)
        klast = cpw - 1
        wait_g(1)
        fire_s(klast, 1)
        wait_s(klast - 1, 0)
        wait_s(klast, 1)
        plsc.subcore_barrier()
        for t in range(RPT // 128):
            r0 = sid * RPT + t * 128
            pltpu.sync_copy(acc.at[pl.ds(r0, 128)],
                            out_hbm.at[cid, pl.ds(r0, 128)])

    return _agg


_agg_h1 = _make_agg(H1)
_agg_h2 = _make_agg(H2)


RB = 1000

def _dinv_from(deg_blk):
    d = deg_blk[:, 0:1] + deg_blk[:, 1:2] + 1.0
    return lax.rsqrt(d)


def _tc_a_body(deg_ref, x_ref, w1_ref, hs_ref):
    dinv = _dinv_from(deg_ref[...])
    h = jnp.dot(x_ref[...], w1_ref[...], preferred_element_type=jnp.float32)
    hs_ref[...] = h * dinv


def _tc_b_body(deg_ref, agg_ref, hs1_ref, w2_ref, b1_ref, hs2_ref):
    dinv = _dinv_from(deg_ref[...])
    agg = agg_ref[0] + agg_ref[1]
    z = jnp.maximum(dinv * (agg + hs1_ref[...]) + b1_ref[...], 0.0)
    h2 = jnp.dot(z, w2_ref[...], preferred_element_type=jnp.float32)
    hs2_ref[...] = h2 * dinv


def _tc_c_body(deg_ref, agg_ref, hs2_ref, wfc_ref, b2_ref, bfc_ref, out_ref):
    dinv = _dinv_from(deg_ref[...])
    agg = agg_ref[0] + agg_ref[1]
    z = jnp.maximum(dinv * (agg + hs2_ref[...]) + b2_ref[...], 0.0)
    o = jnp.dot(z, wfc_ref[...], preferred_element_type=jnp.float32)
    out_ref[...] = 1.0 / (1.0 + jnp.exp(-(o + bfc_ref[...])))


_G = N // RB

_tc_a = pl.pallas_call(
    _tc_a_body,
    grid=(_G,),
    in_specs=[
        pl.BlockSpec((RB, 2), lambda i: (i, 0)),
        pl.BlockSpec((RB, D_IN), lambda i: (i, 0)),
        pl.BlockSpec((D_IN, H1), lambda i: (0, 0)),
    ],
    out_specs=pl.BlockSpec((RB, H1), lambda i: (i, 0)),
    out_shape=jax.ShapeDtypeStruct((N, H1), jnp.float32),
)

_tc_b = pl.pallas_call(
    _tc_b_body,
    grid=(_G,),
    in_specs=[
        pl.BlockSpec((RB, 2), lambda i: (i, 0)),
        pl.BlockSpec((NC, RB, H1), lambda i: (0, i, 0)),
        pl.BlockSpec((RB, H1), lambda i: (i, 0)),
        pl.BlockSpec((H1, H2), lambda i: (0, 0)),
        pl.BlockSpec((1, H1), lambda i: (0, 0)),
    ],
    out_specs=pl.BlockSpec((RB, H2), lambda i: (i, 0)),
    out_shape=jax.ShapeDtypeStruct((N, H2), jnp.float32),
)

_tc_c = pl.pallas_call(
    _tc_c_body,
    grid=(_G,),
    in_specs=[
        pl.BlockSpec((RB, 2), lambda i: (i, 0)),
        pl.BlockSpec((NC, RB, H2), lambda i: (0, i, 0)),
        pl.BlockSpec((RB, H2), lambda i: (i, 0)),
        pl.BlockSpec((H2, 1), lambda i: (0, 0)),
        pl.BlockSpec((1, H2), lambda i: (0, 0)),
        pl.BlockSpec((1, 1), lambda i: (0, 0)),
    ],
    out_specs=pl.BlockSpec((RB, 1), lambda i: (i, 0)),
    out_shape=jax.ShapeDtypeStruct((N, 1), jnp.float32),
)


def kernel(x, edge_index, W1, b1, W2, b2, Wfc, bfc):
    src = edge_index[0]
    dst = edge_index[1]
    pad_iota = jnp.arange(EPL - E, dtype=jnp.int32)
    pad_src = pad_iota % N
    pad_dst = N + pad_iota % (NP - N)
    src2d = jnp.concatenate([src, pad_src]).reshape(EPL // IW, IW)
    dst2d = jnp.concatenate([dst, pad_dst]).reshape(EPL // IW, IW)

    degp = _deg_kernel(dst2d)
    degT = degp.T

    hs1 = _tc_a(degT, x, W1)
    agg1 = _agg_h1(hs1, src2d, dst2d)
    hs2 = _tc_b(degT, agg1, hs1, W2, b1.reshape(1, H1))
    agg2 = _agg_h2(hs2, src2d, dst2d)
    out = _tc_c(degT, agg2, hs2, Wfc, b2.reshape(1, H2), bfc.reshape(1, 1))
    return out
